# Initial kernel scaffold; baseline (speedup 1.0000x reference)
import jax
import jax.numpy as jnp
from jax.experimental import pallas as pl


def kernel(z_e_x, embedding_weight):
    raise NotImplementedError("write your pallas kernel here")


# speedup vs baseline: 1.0554x; 1.0554x over previous
import jax
import jax.numpy as jnp
from jax.experimental import pallas as pl

K_CB = 1024
D_CB = 256


def _vq_kernel(x_ref, cb_ref, out_ref):
    x = x_ref[0]
    cb = cb_ref[...]
    mm = jnp.dot(cb, x, preferred_element_type=jnp.float32)
    cnorm = jnp.sum(cb * cb, axis=1, keepdims=True)
    fnorm = jnp.sum(x * x, axis=0, keepdims=True)
    dist = (fnorm + cnorm) - 2.0 * mm
    minv = jnp.min(dist, axis=0, keepdims=True)
    kio = jax.lax.broadcasted_iota(jnp.int32, dist.shape, 0)
    idx = jnp.min(jnp.where(dist == minv, kio, K_CB), axis=0)
    out_ref[0, 0, :] = idx.astype(jnp.int32)


def kernel(z_e_x, embedding_weight):
    B, D, H, W = z_e_x.shape
    hw = H * W
    x = z_e_x.reshape(B, D, hw)
    out = pl.pallas_call(
        _vq_kernel,
        grid=(B,),
        in_specs=[
            pl.BlockSpec((1, D, hw), lambda b: (b, 0, 0)),
            pl.BlockSpec((K_CB, D_CB), lambda b: (0, 0)),
        ],
        out_specs=pl.BlockSpec((1, 1, hw), lambda b: (b, 0, 0)),
        out_shape=jax.ShapeDtypeStruct((B, 1, hw), jnp.int32),
    )(x, embedding_weight)
    return out.reshape(B, H, W)
